# baseline (speedup 1.0000x reference)
import functools

import jax
import jax.numpy as jnp
from jax import lax
from jax.experimental import pallas as pl
from jax.experimental.pallas import tpu as pltpu
from jax.experimental.pallas import tpu_sc as plsc

N_NODES = 100000
MEM_DIM = 128
MSG_DIM = 256
B_TOTAL = 4096
ROW_BLK = 512
LANES = 16
NC, NS = 2, 16
NW = NC * NS
B_PER_W = B_TOTAL // NW
POS_BITS = 12
SENTINEL = 0x7FFFFFFF

_mesh = plsc.VectorSubcoreMesh(core_axis_name="c", subcore_axis_name="s")


@functools.partial(
    pl.kernel,
    mesh=_mesh,
    out_type=(
        jax.ShapeDtypeStruct((B_TOTAL, MEM_DIM), jnp.float32),
        jax.ShapeDtypeStruct((B_TOTAL,), jnp.int32),
    ),
    scratch_types=[
        pltpu.VMEM((B_PER_W,), jnp.int32),
        pltpu.VMEM((B_PER_W, MEM_DIM), jnp.float32),
        pltpu.VMEM((B_TOTAL,), jnp.int32),
        pltpu.VMEM((B_TOTAL,), jnp.int32),
        pltpu.VMEM((N_NODES,), jnp.int32),
        pltpu.VMEM((2 * LANES,), jnp.int32),
        pltpu.SemaphoreType.DMA,
        pltpu.SemaphoreType.DMA,
    ],
    compiler_params=pltpu.CompilerParams(needs_layout_passes=False),
)
def _sc_gather_win(mem_hbm, idx_hbm, out_hbm, win_hbm,
                   idx_v, rows_v, ids_v, win_v, table_v, shift_v, sem, sem2):
    wid = lax.axis_index("s") * NC + lax.axis_index("c")
    base = wid * B_PER_W
    pltpu.sync_copy(idx_hbm.at[pl.ds(base, B_PER_W)], idx_v)
    g = pltpu.async_copy(mem_hbm.at[idx_v], rows_v, sem)

    @pl.when(wid == 0)
    def _win():
        pltpu.sync_copy(idx_hbm, ids_v)
        shift_v[pl.ds(LANES, LANES)] = jnp.full((LANES,), SENTINEL,
                                                dtype=jnp.int32)
        liota = lax.iota(jnp.int32, LANES)

        def _init(it, carry):
            ids16 = ids_v[pl.ds(it * LANES, LANES)]
            plsc.store_scatter(table_v, [ids16], jnp.zeros((LANES,), jnp.int32))
            return carry

        lax.fori_loop(0, B_TOTAL // LANES, _init, 0)

        def _scan(it, carry):
            ids16 = ids_v[pl.ds(it * LANES, LANES)]
            pos = liota + it * LANES
            k = ids16 * (1 << POS_BITS) + pos
            ks, _ = plsc.sort_key_val(k, k)
            shift_v[pl.ds(0, LANES)] = ks
            nxt = plsc.load_gather(shift_v, [liota + 1])
            sid = lax.shift_right_logical(ks, POS_BITS)
            spos = lax.bitwise_and(ks, (1 << POS_BITS) - 1)
            winner = lax.shift_right_logical(nxt, POS_BITS) != sid
            cur = plsc.load_gather(table_v, [sid])
            plsc.store_scatter(table_v, [sid], jnp.maximum(cur, spos),
                               mask=winner)
            return carry

        lax.fori_loop(0, B_TOTAL // LANES, _scan, 0)

        def _readout(it, carry):
            ids16 = ids_v[pl.ds(it * LANES, LANES)]
            win_v[pl.ds(it * LANES, LANES)] = plsc.load_gather(table_v, [ids16])
            return carry

        lax.fori_loop(0, B_TOTAL // LANES, _readout, 0)
        pltpu.sync_copy(win_v, win_hbm)

    g.wait()
    pltpu.sync_copy(rows_v, out_hbm.at[pl.ds(base, B_PER_W)])


@functools.partial(
    pl.kernel,
    mesh=_mesh,
    out_type=(),
    scratch_types=[
        pltpu.VMEM((B_PER_W,), jnp.int32),
        pltpu.VMEM((B_PER_W,), jnp.int32),
        pltpu.VMEM((B_PER_W, MEM_DIM), jnp.float32),
        pltpu.VMEM((B_PER_W,), jnp.float32),
        pltpu.SemaphoreType.DMA,
        pltpu.SemaphoreType.DMA,
        pltpu.SemaphoreType.DMA,
        pltpu.SemaphoreType.DMA,
    ],
)
def _sc_scatter(mem_ref, lu_ref, idx_hbm, win_hbm, upd_hbm, ts_hbm,
                idx_v, win_v, rows_v, ts_v, sem_a, sem_b, sem_c, sem_d):
    wid = lax.axis_index("s") * NC + lax.axis_index("c")
    base = wid * B_PER_W
    pltpu.sync_copy(idx_hbm.at[pl.ds(base, B_PER_W)], idx_v)
    pltpu.sync_copy(win_hbm.at[pl.ds(base, B_PER_W)], win_v)
    g_rows = pltpu.async_copy(upd_hbm.at[win_v], rows_v, sem_a)
    g_ts = pltpu.async_copy(ts_hbm.at[win_v], ts_v, sem_b)
    g_rows.wait()
    s_rows = pltpu.async_copy(rows_v, mem_ref.at[idx_v], sem_c)
    g_ts.wait()
    s_ts = pltpu.async_copy(ts_v, lu_ref.at[idx_v], sem_d)
    s_rows.wait()
    s_ts.wait()


def _gru_body(msg_ref, h_ref, wih_ref, whh_ref, bih_ref, bhh_ref, upd_ref):
    x = msg_ref[...]
    h = h_ref[...]
    gi = lax.dot_general(x, wih_ref[...], (((1,), (1,)), ((), ())),
                         preferred_element_type=jnp.float32) + bih_ref[...]
    gh = lax.dot_general(h, whh_ref[...], (((1,), (1,)), ((), ())),
                         preferred_element_type=jnp.float32) + bhh_ref[...]
    i_r = gi[:, 0 * MEM_DIM:1 * MEM_DIM]
    i_z = gi[:, 1 * MEM_DIM:2 * MEM_DIM]
    i_n = gi[:, 2 * MEM_DIM:3 * MEM_DIM]
    h_r = gh[:, 0 * MEM_DIM:1 * MEM_DIM]
    h_z = gh[:, 1 * MEM_DIM:2 * MEM_DIM]
    h_n = gh[:, 2 * MEM_DIM:3 * MEM_DIM]
    r = jax.nn.sigmoid(i_r + h_r)
    z = jax.nn.sigmoid(i_z + h_z)
    n = jnp.tanh(i_n + r * h_n)
    upd_ref[...] = (1.0 - z) * n + z * h


def _gru(msgs, h, W_ih, W_hh, b_ih, b_hh):
    grid = (B_TOTAL // ROW_BLK,)
    return pl.pallas_call(
        _gru_body,
        grid=grid,
        in_specs=[
            pl.BlockSpec((ROW_BLK, MSG_DIM), lambda i: (i, 0)),
            pl.BlockSpec((ROW_BLK, MEM_DIM), lambda i: (i, 0)),
            pl.BlockSpec((3 * MEM_DIM, MSG_DIM), lambda i: (0, 0)),
            pl.BlockSpec((3 * MEM_DIM, MEM_DIM), lambda i: (0, 0)),
            pl.BlockSpec((1, 3 * MEM_DIM), lambda i: (0, 0)),
            pl.BlockSpec((1, 3 * MEM_DIM), lambda i: (0, 0)),
        ],
        out_specs=pl.BlockSpec((ROW_BLK, MEM_DIM), lambda i: (i, 0)),
        out_shape=jax.ShapeDtypeStruct((B_TOTAL, MEM_DIM), jnp.float32),
    )(msgs, h, W_ih, W_hh, b_ih.reshape(1, -1), b_hh.reshape(1, -1))


def kernel(memory, last_update, unique_node_ids, unique_messages, timestamps,
           W_ih, W_hh, b_ih, b_hh):
    ids = unique_node_ids
    h, win = _sc_gather_win(memory, ids)
    upd = _gru(unique_messages, h, W_ih, W_hh, b_ih, b_hh)
    updated_memory = memory.at[ids].set(jnp.take(upd, win, axis=0))
    updated_last_update = last_update.at[ids].set(timestamps)
    return updated_memory, updated_last_update


# speedup vs baseline: 2.0499x; 2.0499x over previous
import functools

import jax
import jax.numpy as jnp
from jax import lax
from jax.experimental import pallas as pl
from jax.experimental.pallas import tpu as pltpu
from jax.experimental.pallas import tpu_sc as plsc

N_NODES = 100000
MEM_DIM = 128
MSG_DIM = 256
B_TOTAL = 4096
ROW_BLK = 1024
LANES = 16
NC, NS = 2, 16
NW = NC * NS
B_PER_W = B_TOTAL // NW
POS_BITS = 12
SENTINEL = 0x7FFFFFFF
UNROLL = 4

_mesh = plsc.VectorSubcoreMesh(core_axis_name="c", subcore_axis_name="s")


@functools.partial(
    pl.kernel,
    mesh=_mesh,
    out_type=jax.ShapeDtypeStruct((B_TOTAL, MEM_DIM), jnp.float32),
    scratch_types=[
        pltpu.VMEM((B_PER_W,), jnp.int32),
        pltpu.VMEM((B_PER_W, MEM_DIM), jnp.float32),
        pltpu.SemaphoreType.DMA,
    ],
)
def _sc_gather(mem_hbm, idx_hbm, out_hbm, idx_v, rows_v, sem):
    wid = lax.axis_index("s") * NC + lax.axis_index("c")
    base = wid * B_PER_W
    pltpu.sync_copy(idx_hbm.at[pl.ds(base, B_PER_W)], idx_v)
    pltpu.async_copy(mem_hbm.at[idx_v], rows_v, sem).wait()
    pltpu.sync_copy(rows_v, out_hbm.at[pl.ds(base, B_PER_W)])


@functools.partial(
    pl.kernel,
    mesh=_mesh,
    out_type=jax.ShapeDtypeStruct((B_TOTAL,), jnp.int32),
    scratch_types=[
        pltpu.VMEM((B_TOTAL,), jnp.int32),
        pltpu.VMEM((B_TOTAL,), jnp.int32),
        pltpu.VMEM((N_NODES,), jnp.int32),
        pltpu.VMEM((2 * LANES,), jnp.int32),
    ],
    compiler_params=pltpu.CompilerParams(needs_layout_passes=False),
)
def _sc_win(idx_hbm, win_hbm, ids_v, win_v, table_v, shift_v):
    wid = lax.axis_index("s") * NC + lax.axis_index("c")

    @pl.when(wid == 0)
    def _win():
        pltpu.sync_copy(idx_hbm, ids_v)
        shift_v[pl.ds(LANES, LANES)] = jnp.full((LANES,), SENTINEL,
                                                dtype=jnp.int32)
        liota = lax.iota(jnp.int32, LANES)

        def _init(it, carry):
            for u in range(UNROLL):
                ids16 = ids_v[pl.ds((it * UNROLL + u) * LANES, LANES)]
                plsc.store_scatter(table_v, [ids16],
                                   jnp.zeros((LANES,), jnp.int32))
            return carry

        lax.fori_loop(0, B_TOTAL // (LANES * UNROLL), _init, 0)

        def _scan(it, carry):
            for u in range(UNROLL):
                vi = it * UNROLL + u
                ids16 = ids_v[pl.ds(vi * LANES, LANES)]
                pos = liota + vi * LANES
                k = ids16 * (1 << POS_BITS) + pos
                ks, _ = plsc.sort_key_val(k, k)
                shift_v[pl.ds(0, LANES)] = ks
                nxt = plsc.load_gather(shift_v, [liota + 1])
                sid = lax.shift_right_logical(ks, POS_BITS)
                spos = lax.bitwise_and(ks, (1 << POS_BITS) - 1)
                winner = lax.shift_right_logical(nxt, POS_BITS) != sid
                cur = plsc.load_gather(table_v, [sid])
                plsc.store_scatter(table_v, [sid], jnp.maximum(cur, spos),
                                   mask=winner)
            return carry

        lax.fori_loop(0, B_TOTAL // (LANES * UNROLL), _scan, 0)

        def _readout(it, carry):
            for u in range(UNROLL):
                vi = it * UNROLL + u
                ids16 = ids_v[pl.ds(vi * LANES, LANES)]
                win_v[pl.ds(vi * LANES, LANES)] = plsc.load_gather(
                    table_v, [ids16])
            return carry

        lax.fori_loop(0, B_TOTAL // (LANES * UNROLL), _readout, 0)
        pltpu.sync_copy(win_v, win_hbm)


@functools.partial(
    pl.kernel,
    mesh=_mesh,
    out_type=(),
    scratch_types=[
        pltpu.VMEM((B_PER_W,), jnp.int32),
        pltpu.VMEM((B_PER_W,), jnp.int32),
        pltpu.VMEM((B_PER_W, MEM_DIM), jnp.float32),
        pltpu.VMEM((B_PER_W,), jnp.float32),
        pltpu.SemaphoreType.DMA,
        pltpu.SemaphoreType.DMA,
        pltpu.SemaphoreType.DMA,
        pltpu.SemaphoreType.DMA,
    ],
)
def _sc_scatter(mem_ref, lu_ref, idx_hbm, win_hbm, upd_hbm, ts_hbm,
                idx_v, win_v, rows_v, ts_v, sem_a, sem_b, sem_c, sem_d):
    wid = lax.axis_index("s") * NC + lax.axis_index("c")
    base = wid * B_PER_W
    c_idx = pltpu.async_copy(idx_hbm.at[pl.ds(base, B_PER_W)], idx_v, sem_a)
    c_win = pltpu.async_copy(win_hbm.at[pl.ds(base, B_PER_W)], win_v, sem_b)
    c_idx.wait()
    c_win.wait()
    g_rows = pltpu.async_copy(upd_hbm.at[win_v], rows_v, sem_a)
    g_ts = pltpu.async_copy(ts_hbm.at[win_v], ts_v, sem_b)
    g_rows.wait()
    s_rows = pltpu.async_copy(rows_v, mem_ref.at[idx_v], sem_c)
    g_ts.wait()
    s_ts = pltpu.async_copy(ts_v, lu_ref.at[idx_v], sem_d)
    s_rows.wait()
    s_ts.wait()


def _gru_body(msg_ref, h_ref, wih_ref, whh_ref, bih_ref, bhh_ref, upd_ref):
    x = msg_ref[...]
    h = h_ref[...]
    gi = lax.dot_general(x, wih_ref[...], (((1,), (1,)), ((), ())),
                         preferred_element_type=jnp.float32) + bih_ref[...]
    gh = lax.dot_general(h, whh_ref[...], (((1,), (1,)), ((), ())),
                         preferred_element_type=jnp.float32) + bhh_ref[...]
    i_r = gi[:, 0 * MEM_DIM:1 * MEM_DIM]
    i_z = gi[:, 1 * MEM_DIM:2 * MEM_DIM]
    i_n = gi[:, 2 * MEM_DIM:3 * MEM_DIM]
    h_r = gh[:, 0 * MEM_DIM:1 * MEM_DIM]
    h_z = gh[:, 1 * MEM_DIM:2 * MEM_DIM]
    h_n = gh[:, 2 * MEM_DIM:3 * MEM_DIM]
    r = jax.nn.sigmoid(i_r + h_r)
    z = jax.nn.sigmoid(i_z + h_z)
    n = jnp.tanh(i_n + r * h_n)
    upd_ref[...] = (1.0 - z) * n + z * h


def _gru(msgs, h, W_ih, W_hh, b_ih, b_hh):
    grid = (B_TOTAL // ROW_BLK,)
    return pl.pallas_call(
        _gru_body,
        grid=grid,
        in_specs=[
            pl.BlockSpec((ROW_BLK, MSG_DIM), lambda i: (i, 0)),
            pl.BlockSpec((ROW_BLK, MEM_DIM), lambda i: (i, 0)),
            pl.BlockSpec((3 * MEM_DIM, MSG_DIM), lambda i: (0, 0)),
            pl.BlockSpec((3 * MEM_DIM, MEM_DIM), lambda i: (0, 0)),
            pl.BlockSpec((3 * MEM_DIM,), lambda i: (0,)),
            pl.BlockSpec((3 * MEM_DIM,), lambda i: (0,)),
        ],
        out_specs=pl.BlockSpec((ROW_BLK, MEM_DIM), lambda i: (i, 0)),
        out_shape=jax.ShapeDtypeStruct((B_TOTAL, MEM_DIM), jnp.float32),
    )(msgs, h, W_ih, W_hh, b_ih, b_hh)


def kernel(memory, last_update, unique_node_ids, unique_messages, timestamps,
           W_ih, W_hh, b_ih, b_hh):
    ids = unique_node_ids
    h = _sc_gather(memory, ids)
    win = _sc_win(ids)
    upd = _gru(unique_messages, h, W_ih, W_hh, b_ih, b_hh)
    mem_ref = jax.new_ref(memory)
    lu_ref = jax.new_ref(last_update)
    _sc_scatter(mem_ref, lu_ref, ids, win, upd, timestamps)
    return mem_ref[...], lu_ref[...]


# speedup vs baseline: 2.0968x; 1.0229x over previous
import functools

import jax
import jax.numpy as jnp
from jax import lax
from jax.experimental import pallas as pl
from jax.experimental.pallas import tpu as pltpu
from jax.experimental.pallas import tpu_sc as plsc

N_NODES = 100000
MEM_DIM = 128
MSG_DIM = 256
B_TOTAL = 4096
ROW_BLK = 1024
LANES = 16
NC, NS = 2, 16
NW = NC * NS
B_PER_W = B_TOTAL // NW
POS_BITS = 12
SENTINEL = 0x7FFFFFFF
UNROLL = 4

_mesh = plsc.VectorSubcoreMesh(core_axis_name="c", subcore_axis_name="s")


@functools.partial(
    pl.kernel,
    mesh=_mesh,
    out_type=jax.ShapeDtypeStruct((B_TOTAL, MEM_DIM), jnp.float32),
    scratch_types=[
        pltpu.VMEM((B_PER_W,), jnp.int32),
        pltpu.VMEM((B_PER_W, MEM_DIM), jnp.float32),
        pltpu.SemaphoreType.DMA,
    ],
)
def _sc_gather(mem_hbm, idx_hbm, out_hbm, idx_v, rows_v, sem):
    wid = lax.axis_index("s") * NC + lax.axis_index("c")
    base = wid * B_PER_W
    pltpu.sync_copy(idx_hbm.at[pl.ds(base, B_PER_W)], idx_v)
    pltpu.async_copy(mem_hbm.at[idx_v], rows_v, sem).wait()
    pltpu.sync_copy(rows_v, out_hbm.at[pl.ds(base, B_PER_W)])


@functools.partial(
    pl.kernel,
    mesh=_mesh,
    out_type=jax.ShapeDtypeStruct((B_TOTAL,), jnp.int32),
    scratch_types=[
        pltpu.VMEM((B_TOTAL,), jnp.int32),
        pltpu.VMEM((B_TOTAL,), jnp.int32),
        pltpu.VMEM((N_NODES,), jnp.int32),
        pltpu.VMEM((2 * LANES,), jnp.int32),
    ],
    compiler_params=pltpu.CompilerParams(needs_layout_passes=False),
)
def _sc_win(idx_hbm, win_hbm, ids_v, win_v, table_v, shift_v):
    wid = lax.axis_index("s") * NC + lax.axis_index("c")

    @pl.when(wid == 0)
    def _win():
        pltpu.sync_copy(idx_hbm, ids_v)
        shift_v[pl.ds(LANES, LANES)] = jnp.full((LANES,), SENTINEL,
                                                dtype=jnp.int32)
        liota = lax.iota(jnp.int32, LANES)

        def _init(it, carry):
            for u in range(UNROLL):
                ids16 = ids_v[pl.ds((it * UNROLL + u) * LANES, LANES)]
                plsc.store_scatter(table_v, [ids16],
                                   jnp.zeros((LANES,), jnp.int32))
            return carry

        lax.fori_loop(0, B_TOTAL // (LANES * UNROLL), _init, 0)

        def _scan(it, carry):
            for u in range(UNROLL):
                vi = it * UNROLL + u
                ids16 = ids_v[pl.ds(vi * LANES, LANES)]
                pos = liota + vi * LANES
                k = ids16 * (1 << POS_BITS) + pos
                ks, _ = plsc.sort_key_val(k, k)
                shift_v[pl.ds(0, LANES)] = ks
                nxt = plsc.load_gather(shift_v, [liota + 1])
                sid = lax.shift_right_logical(ks, POS_BITS)
                spos = lax.bitwise_and(ks, (1 << POS_BITS) - 1)
                winner = lax.shift_right_logical(nxt, POS_BITS) != sid
                cur = plsc.load_gather(table_v, [sid])
                plsc.store_scatter(table_v, [sid], jnp.maximum(cur, spos),
                                   mask=winner)
            return carry

        lax.fori_loop(0, B_TOTAL // (LANES * UNROLL), _scan, 0)

        def _readout(it, carry):
            for u in range(UNROLL):
                vi = it * UNROLL + u
                ids16 = ids_v[pl.ds(vi * LANES, LANES)]
                win_v[pl.ds(vi * LANES, LANES)] = plsc.load_gather(
                    table_v, [ids16])
            return carry

        lax.fori_loop(0, B_TOTAL // (LANES * UNROLL), _readout, 0)
        pltpu.sync_copy(win_v, win_hbm)


@functools.partial(
    pl.kernel,
    mesh=_mesh,
    out_type=(),
    scratch_types=[
        pltpu.VMEM((B_PER_W,), jnp.int32),
        pltpu.VMEM((B_PER_W,), jnp.int32),
        pltpu.VMEM((B_PER_W, MEM_DIM), jnp.float32),
        pltpu.VMEM((B_PER_W,), jnp.float32),
        pltpu.SemaphoreType.DMA,
        pltpu.SemaphoreType.DMA,
        pltpu.SemaphoreType.DMA,
        pltpu.SemaphoreType.DMA,
    ],
)
def _sc_scatter(mem_ref, lu_ref, idx_hbm, win_hbm, upd_hbm, ts_hbm,
                idx_v, win_v, rows_v, ts_v, sem_a, sem_b, sem_c, sem_d):
    wid = lax.axis_index("s") * NC + lax.axis_index("c")
    base = wid * B_PER_W
    c_idx = pltpu.async_copy(idx_hbm.at[pl.ds(base, B_PER_W)], idx_v, sem_a)
    c_win = pltpu.async_copy(win_hbm.at[pl.ds(base, B_PER_W)], win_v, sem_b)
    c_idx.wait()
    c_win.wait()
    g_rows = pltpu.async_copy(upd_hbm.at[win_v], rows_v, sem_a)
    g_ts = pltpu.async_copy(ts_hbm.at[win_v], ts_v, sem_b)
    g_rows.wait()
    s_rows = pltpu.async_copy(rows_v, mem_ref.at[idx_v], sem_c)
    g_ts.wait()
    s_ts = pltpu.async_copy(ts_v, lu_ref.at[idx_v], sem_d)
    s_rows.wait()
    s_ts.wait()


COPY_BLK = 5000


def _copy_body(dep_ref, src_ref, out_ref):
    out_ref[...] = src_ref[...]


def _tc_copy(memory, h):
    grid = (N_NODES // COPY_BLK,)
    return pl.pallas_call(
        _copy_body,
        grid=grid,
        in_specs=[
            pl.BlockSpec((8, MEM_DIM), lambda i: (0, 0)),
            pl.BlockSpec((COPY_BLK, MEM_DIM), lambda i: (i, 0)),
        ],
        out_specs=pl.BlockSpec((COPY_BLK, MEM_DIM), lambda i: (i, 0)),
        out_shape=jax.ShapeDtypeStruct((N_NODES, MEM_DIM), jnp.float32),
    )(h, memory)


def _gru_body(msg_ref, h_ref, wih_ref, whh_ref, bih_ref, bhh_ref, upd_ref):
    x = msg_ref[...]
    h = h_ref[...]
    gi = lax.dot_general(x, wih_ref[...], (((1,), (1,)), ((), ())),
                         preferred_element_type=jnp.float32) + bih_ref[...]
    gh = lax.dot_general(h, whh_ref[...], (((1,), (1,)), ((), ())),
                         preferred_element_type=jnp.float32) + bhh_ref[...]
    i_r = gi[:, 0 * MEM_DIM:1 * MEM_DIM]
    i_z = gi[:, 1 * MEM_DIM:2 * MEM_DIM]
    i_n = gi[:, 2 * MEM_DIM:3 * MEM_DIM]
    h_r = gh[:, 0 * MEM_DIM:1 * MEM_DIM]
    h_z = gh[:, 1 * MEM_DIM:2 * MEM_DIM]
    h_n = gh[:, 2 * MEM_DIM:3 * MEM_DIM]
    r = jax.nn.sigmoid(i_r + h_r)
    z = jax.nn.sigmoid(i_z + h_z)
    n = jnp.tanh(i_n + r * h_n)
    upd_ref[...] = (1.0 - z) * n + z * h


def _gru(msgs, h, W_ih, W_hh, b_ih, b_hh):
    grid = (B_TOTAL // ROW_BLK,)
    return pl.pallas_call(
        _gru_body,
        grid=grid,
        in_specs=[
            pl.BlockSpec((ROW_BLK, MSG_DIM), lambda i: (i, 0)),
            pl.BlockSpec((ROW_BLK, MEM_DIM), lambda i: (i, 0)),
            pl.BlockSpec((3 * MEM_DIM, MSG_DIM), lambda i: (0, 0)),
            pl.BlockSpec((3 * MEM_DIM, MEM_DIM), lambda i: (0, 0)),
            pl.BlockSpec((3 * MEM_DIM,), lambda i: (0,)),
            pl.BlockSpec((3 * MEM_DIM,), lambda i: (0,)),
        ],
        out_specs=pl.BlockSpec((ROW_BLK, MEM_DIM), lambda i: (i, 0)),
        out_shape=jax.ShapeDtypeStruct((B_TOTAL, MEM_DIM), jnp.float32),
    )(msgs, h, W_ih, W_hh, b_ih, b_hh)


def kernel(memory, last_update, unique_node_ids, unique_messages, timestamps,
           W_ih, W_hh, b_ih, b_hh):
    ids = unique_node_ids
    h = _sc_gather(memory, ids)
    win = _sc_win(ids)
    upd = _gru(unique_messages, h, W_ih, W_hh, b_ih, b_hh)
    mem_ref = jax.new_ref(_tc_copy(memory, h))
    lu_ref = jax.new_ref(last_update)
    _sc_scatter(mem_ref, lu_ref, ids, win, upd, timestamps)
    return mem_ref[...], lu_ref[...]


# speedup vs baseline: 2.1635x; 1.0318x over previous
import functools

import jax
import jax.numpy as jnp
from jax import lax
from jax.experimental import pallas as pl
from jax.experimental.pallas import tpu as pltpu
from jax.experimental.pallas import tpu_sc as plsc

N_NODES = 100000
MEM_DIM = 128
MSG_DIM = 256
B_TOTAL = 4096
ROW_BLK = 1024
LANES = 16
NC, NS = 2, 16
NW = NC * NS
B_PER_W = B_TOTAL // NW
POS_BITS = 12
SENTINEL = 0x7FFFFFFF
UNROLL = 4

_mesh = plsc.VectorSubcoreMesh(core_axis_name="c", subcore_axis_name="s")


@functools.partial(
    pl.kernel,
    mesh=_mesh,
    out_type=jax.ShapeDtypeStruct((B_TOTAL, MEM_DIM), jnp.float32),
    scratch_types=[
        pltpu.VMEM((B_PER_W,), jnp.int32),
        pltpu.VMEM((B_PER_W, MEM_DIM), jnp.float32),
        pltpu.SemaphoreType.DMA,
    ],
)
def _sc_gather(mem_hbm, idx_hbm, out_hbm, idx_v, rows_v, sem):
    wid = lax.axis_index("s") * NC + lax.axis_index("c")
    base = wid * B_PER_W
    pltpu.sync_copy(idx_hbm.at[pl.ds(base, B_PER_W)], idx_v)
    pltpu.async_copy(mem_hbm.at[idx_v], rows_v, sem).wait()
    pltpu.sync_copy(rows_v, out_hbm.at[pl.ds(base, B_PER_W)])


@functools.partial(
    pl.kernel,
    mesh=_mesh,
    out_type=jax.ShapeDtypeStruct((B_TOTAL,), jnp.int32),
    scratch_types=[
        pltpu.VMEM((B_TOTAL,), jnp.int32),
        pltpu.VMEM((B_TOTAL,), jnp.int32),
        pltpu.VMEM((N_NODES,), jnp.int32),
        pltpu.VMEM((2 * LANES,), jnp.int32),
    ],
    compiler_params=pltpu.CompilerParams(needs_layout_passes=False),
)
def _sc_win(idx_hbm, win_hbm, ids_v, win_v, table_v, shift_v):
    wid = lax.axis_index("s") * NC + lax.axis_index("c")

    @pl.when(wid == 0)
    def _win():
        pltpu.sync_copy(idx_hbm, ids_v)
        shift_v[pl.ds(LANES, LANES)] = jnp.full((LANES,), SENTINEL,
                                                dtype=jnp.int32)
        liota = lax.iota(jnp.int32, LANES)

        def _init(it, carry):
            for u in range(UNROLL):
                ids16 = ids_v[pl.ds((it * UNROLL + u) * LANES, LANES)]
                plsc.store_scatter(table_v, [ids16],
                                   jnp.zeros((LANES,), jnp.int32))
            return carry

        lax.fori_loop(0, B_TOTAL // (LANES * UNROLL), _init, 0)

        def _scan(it, carry):
            for u in range(UNROLL):
                vi = it * UNROLL + u
                ids16 = ids_v[pl.ds(vi * LANES, LANES)]
                pos = liota + vi * LANES
                k = ids16 * (1 << POS_BITS) + pos
                ks, _ = plsc.sort_key_val(k, k)
                shift_v[pl.ds(0, LANES)] = ks
                nxt = plsc.load_gather(shift_v, [liota + 1])
                sid = lax.shift_right_logical(ks, POS_BITS)
                spos = lax.bitwise_and(ks, (1 << POS_BITS) - 1)
                winner = lax.shift_right_logical(nxt, POS_BITS) != sid
                cur = plsc.load_gather(table_v, [sid])
                plsc.store_scatter(table_v, [sid], jnp.maximum(cur, spos),
                                   mask=winner)
            return carry

        lax.fori_loop(0, B_TOTAL // (LANES * UNROLL), _scan, 0)

        def _readout(it, carry):
            for u in range(UNROLL):
                vi = it * UNROLL + u
                ids16 = ids_v[pl.ds(vi * LANES, LANES)]
                win_v[pl.ds(vi * LANES, LANES)] = plsc.load_gather(
                    table_v, [ids16])
            return carry

        lax.fori_loop(0, B_TOTAL // (LANES * UNROLL), _readout, 0)
        pltpu.sync_copy(win_v, win_hbm)


@functools.partial(
    pl.kernel,
    mesh=_mesh,
    out_type=(),
    scratch_types=[
        pltpu.VMEM((B_PER_W,), jnp.int32),
        pltpu.VMEM((B_PER_W,), jnp.int32),
        pltpu.VMEM((B_PER_W, MEM_DIM), jnp.float32),
        pltpu.VMEM((B_PER_W,), jnp.float32),
        pltpu.SemaphoreType.DMA,
        pltpu.SemaphoreType.DMA,
        pltpu.SemaphoreType.DMA,
        pltpu.SemaphoreType.DMA,
    ],
)
def _sc_scatter(mem_ref, lu_ref, idx_hbm, win_hbm, upd_hbm, ts_hbm,
                idx_v, win_v, rows_v, ts_v, sem_a, sem_b, sem_c, sem_d):
    wid = lax.axis_index("s") * NC + lax.axis_index("c")
    base = wid * B_PER_W
    c_idx = pltpu.async_copy(idx_hbm.at[pl.ds(base, B_PER_W)], idx_v, sem_a)
    c_win = pltpu.async_copy(win_hbm.at[pl.ds(base, B_PER_W)], win_v, sem_b)
    c_idx.wait()
    c_win.wait()
    g_rows = pltpu.async_copy(upd_hbm.at[win_v], rows_v, sem_a)
    g_ts = pltpu.async_copy(ts_hbm.at[win_v], ts_v, sem_b)
    g_rows.wait()
    s_rows = pltpu.async_copy(rows_v, mem_ref.at[idx_v], sem_c)
    g_ts.wait()
    s_ts = pltpu.async_copy(ts_v, lu_ref.at[idx_v], sem_d)
    s_rows.wait()
    s_ts.wait()


COPY_BLK = 10000


def _copy_body(dep_ref, src_ref, out_ref):
    out_ref[...] = src_ref[...]


def _tc_copy(memory, h):
    grid = (N_NODES // COPY_BLK,)
    return pl.pallas_call(
        _copy_body,
        grid=grid,
        in_specs=[
            pl.BlockSpec((8, MEM_DIM), lambda i: (0, 0)),
            pl.BlockSpec((COPY_BLK, MEM_DIM), lambda i: (i, 0)),
        ],
        out_specs=pl.BlockSpec((COPY_BLK, MEM_DIM), lambda i: (i, 0)),
        out_shape=jax.ShapeDtypeStruct((N_NODES, MEM_DIM), jnp.float32),
    )(h, memory)


def _gru_body(msg_ref, h_ref, wih_ref, whh_ref, bih_ref, bhh_ref, upd_ref):
    x = msg_ref[...]
    h = h_ref[...]
    gi = lax.dot_general(x, wih_ref[...], (((1,), (1,)), ((), ())),
                         preferred_element_type=jnp.float32) + bih_ref[...]
    gh = lax.dot_general(h, whh_ref[...], (((1,), (1,)), ((), ())),
                         preferred_element_type=jnp.float32) + bhh_ref[...]
    i_r = gi[:, 0 * MEM_DIM:1 * MEM_DIM]
    i_z = gi[:, 1 * MEM_DIM:2 * MEM_DIM]
    i_n = gi[:, 2 * MEM_DIM:3 * MEM_DIM]
    h_r = gh[:, 0 * MEM_DIM:1 * MEM_DIM]
    h_z = gh[:, 1 * MEM_DIM:2 * MEM_DIM]
    h_n = gh[:, 2 * MEM_DIM:3 * MEM_DIM]
    r = jax.nn.sigmoid(i_r + h_r)
    z = jax.nn.sigmoid(i_z + h_z)
    n = jnp.tanh(i_n + r * h_n)
    upd_ref[...] = (1.0 - z) * n + z * h


def _gru(msgs, h, W_ih, W_hh, b_ih, b_hh):
    grid = (B_TOTAL // ROW_BLK,)
    return pl.pallas_call(
        _gru_body,
        grid=grid,
        in_specs=[
            pl.BlockSpec((ROW_BLK, MSG_DIM), lambda i: (i, 0)),
            pl.BlockSpec((ROW_BLK, MEM_DIM), lambda i: (i, 0)),
            pl.BlockSpec((3 * MEM_DIM, MSG_DIM), lambda i: (0, 0)),
            pl.BlockSpec((3 * MEM_DIM, MEM_DIM), lambda i: (0, 0)),
            pl.BlockSpec((3 * MEM_DIM,), lambda i: (0,)),
            pl.BlockSpec((3 * MEM_DIM,), lambda i: (0,)),
        ],
        out_specs=pl.BlockSpec((ROW_BLK, MEM_DIM), lambda i: (i, 0)),
        out_shape=jax.ShapeDtypeStruct((B_TOTAL, MEM_DIM), jnp.float32),
    )(msgs, h, W_ih, W_hh, b_ih, b_hh)


def kernel(memory, last_update, unique_node_ids, unique_messages, timestamps,
           W_ih, W_hh, b_ih, b_hh):
    ids = unique_node_ids
    h = _sc_gather(memory, ids)
    win = _sc_win(ids)
    upd = _gru(unique_messages, h, W_ih, W_hh, b_ih, b_hh)
    mem_ref = jax.new_ref(_tc_copy(memory, h))
    lu_ref = jax.new_ref(last_update)
    _sc_scatter(mem_ref, lu_ref, ids, win, upd, timestamps)
    return mem_ref[...], lu_ref[...]


# speedup vs baseline: 2.1928x; 1.0135x over previous
import functools

import jax
import jax.numpy as jnp
from jax import lax
from jax.experimental import pallas as pl
from jax.experimental.pallas import tpu as pltpu
from jax.experimental.pallas import tpu_sc as plsc

N_NODES = 100000
MEM_DIM = 128
MSG_DIM = 256
B_TOTAL = 4096
ROW_BLK = 2048
LANES = 16
NC, NS = 2, 16
NW = NC * NS
B_PER_W = B_TOTAL // NW
POS_BITS = 12
SENTINEL = 0x7FFFFFFF
UNROLL = 4

_mesh = plsc.VectorSubcoreMesh(core_axis_name="c", subcore_axis_name="s")


@functools.partial(
    pl.kernel,
    mesh=_mesh,
    out_type=jax.ShapeDtypeStruct((B_TOTAL, MEM_DIM), jnp.float32),
    scratch_types=[
        pltpu.VMEM((B_PER_W,), jnp.int32),
        pltpu.VMEM((B_PER_W, MEM_DIM), jnp.float32),
        pltpu.SemaphoreType.DMA,
    ],
)
def _sc_gather(mem_hbm, idx_hbm, out_hbm, idx_v, rows_v, sem):
    wid = lax.axis_index("s") * NC + lax.axis_index("c")
    base = wid * B_PER_W
    pltpu.sync_copy(idx_hbm.at[pl.ds(base, B_PER_W)], idx_v)
    pltpu.async_copy(mem_hbm.at[idx_v], rows_v, sem).wait()
    pltpu.sync_copy(rows_v, out_hbm.at[pl.ds(base, B_PER_W)])


@functools.partial(
    pl.kernel,
    mesh=_mesh,
    out_type=jax.ShapeDtypeStruct((B_TOTAL,), jnp.int32),
    scratch_types=[
        pltpu.VMEM((B_TOTAL,), jnp.int32),
        pltpu.VMEM((B_TOTAL,), jnp.int32),
        pltpu.VMEM((N_NODES,), jnp.int32),
        pltpu.VMEM((2 * LANES,), jnp.int32),
    ],
    compiler_params=pltpu.CompilerParams(needs_layout_passes=False),
)
def _sc_win(idx_hbm, win_hbm, ids_v, win_v, table_v, shift_v):
    wid = lax.axis_index("s") * NC + lax.axis_index("c")

    @pl.when(wid == 0)
    def _win():
        pltpu.sync_copy(idx_hbm, ids_v)
        shift_v[pl.ds(LANES, LANES)] = jnp.full((LANES,), SENTINEL,
                                                dtype=jnp.int32)
        liota = lax.iota(jnp.int32, LANES)

        def _init(it, carry):
            for u in range(UNROLL):
                ids16 = ids_v[pl.ds((it * UNROLL + u) * LANES, LANES)]
                plsc.store_scatter(table_v, [ids16],
                                   jnp.zeros((LANES,), jnp.int32))
            return carry

        lax.fori_loop(0, B_TOTAL // (LANES * UNROLL), _init, 0)

        def _scan(it, carry):
            for u in range(UNROLL):
                vi = it * UNROLL + u
                ids16 = ids_v[pl.ds(vi * LANES, LANES)]
                pos = liota + vi * LANES
                k = ids16 * (1 << POS_BITS) + pos
                ks, _ = plsc.sort_key_val(k, k)
                shift_v[pl.ds(0, LANES)] = ks
                nxt = plsc.load_gather(shift_v, [liota + 1])
                sid = lax.shift_right_logical(ks, POS_BITS)
                spos = lax.bitwise_and(ks, (1 << POS_BITS) - 1)
                winner = lax.shift_right_logical(nxt, POS_BITS) != sid
                cur = plsc.load_gather(table_v, [sid])
                plsc.store_scatter(table_v, [sid], jnp.maximum(cur, spos),
                                   mask=winner)
            return carry

        lax.fori_loop(0, B_TOTAL // (LANES * UNROLL), _scan, 0)

        def _readout(it, carry):
            for u in range(UNROLL):
                vi = it * UNROLL + u
                ids16 = ids_v[pl.ds(vi * LANES, LANES)]
                win_v[pl.ds(vi * LANES, LANES)] = plsc.load_gather(
                    table_v, [ids16])
            return carry

        lax.fori_loop(0, B_TOTAL // (LANES * UNROLL), _readout, 0)
        pltpu.sync_copy(win_v, win_hbm)


@functools.partial(
    pl.kernel,
    mesh=_mesh,
    out_type=(),
    scratch_types=[
        pltpu.VMEM((B_PER_W,), jnp.int32),
        pltpu.VMEM((B_PER_W,), jnp.int32),
        pltpu.VMEM((B_PER_W, MEM_DIM), jnp.float32),
        pltpu.VMEM((B_PER_W,), jnp.float32),
        pltpu.SemaphoreType.DMA,
        pltpu.SemaphoreType.DMA,
        pltpu.SemaphoreType.DMA,
        pltpu.SemaphoreType.DMA,
    ],
)
def _sc_scatter(mem_ref, lu_ref, idx_hbm, win_hbm, upd_hbm, ts_hbm,
                idx_v, win_v, rows_v, ts_v, sem_a, sem_b, sem_c, sem_d):
    wid = lax.axis_index("s") * NC + lax.axis_index("c")
    base = wid * B_PER_W
    c_idx = pltpu.async_copy(idx_hbm.at[pl.ds(base, B_PER_W)], idx_v, sem_a)
    c_win = pltpu.async_copy(win_hbm.at[pl.ds(base, B_PER_W)], win_v, sem_b)
    c_idx.wait()
    c_win.wait()
    g_rows = pltpu.async_copy(upd_hbm.at[win_v], rows_v, sem_a)
    g_ts = pltpu.async_copy(ts_hbm.at[win_v], ts_v, sem_b)
    g_rows.wait()
    s_rows = pltpu.async_copy(rows_v, mem_ref.at[idx_v], sem_c)
    g_ts.wait()
    s_ts = pltpu.async_copy(ts_v, lu_ref.at[idx_v], sem_d)
    s_rows.wait()
    s_ts.wait()


COPY_BLK = 20000


def _copy_body(dep_ref, src_ref, out_ref):
    out_ref[...] = src_ref[...]


def _tc_copy(memory, h):
    grid = (N_NODES // COPY_BLK,)
    return pl.pallas_call(
        _copy_body,
        grid=grid,
        in_specs=[
            pl.BlockSpec((8, MEM_DIM), lambda i: (0, 0)),
            pl.BlockSpec((COPY_BLK, MEM_DIM), lambda i: (i, 0)),
        ],
        out_specs=pl.BlockSpec((COPY_BLK, MEM_DIM), lambda i: (i, 0)),
        out_shape=jax.ShapeDtypeStruct((N_NODES, MEM_DIM), jnp.float32),
    )(h, memory)


def _gru_body(msg_ref, h_ref, wih_ref, whh_ref, bih_ref, bhh_ref, upd_ref):
    x = msg_ref[...]
    h = h_ref[...]
    gi = lax.dot_general(x, wih_ref[...], (((1,), (1,)), ((), ())),
                         preferred_element_type=jnp.float32) + bih_ref[...]
    gh = lax.dot_general(h, whh_ref[...], (((1,), (1,)), ((), ())),
                         preferred_element_type=jnp.float32) + bhh_ref[...]
    i_r = gi[:, 0 * MEM_DIM:1 * MEM_DIM]
    i_z = gi[:, 1 * MEM_DIM:2 * MEM_DIM]
    i_n = gi[:, 2 * MEM_DIM:3 * MEM_DIM]
    h_r = gh[:, 0 * MEM_DIM:1 * MEM_DIM]
    h_z = gh[:, 1 * MEM_DIM:2 * MEM_DIM]
    h_n = gh[:, 2 * MEM_DIM:3 * MEM_DIM]
    r = jax.nn.sigmoid(i_r + h_r)
    z = jax.nn.sigmoid(i_z + h_z)
    n = jnp.tanh(i_n + r * h_n)
    upd_ref[...] = (1.0 - z) * n + z * h


def _gru(msgs, h, W_ih, W_hh, b_ih, b_hh):
    grid = (B_TOTAL // ROW_BLK,)
    return pl.pallas_call(
        _gru_body,
        grid=grid,
        in_specs=[
            pl.BlockSpec((ROW_BLK, MSG_DIM), lambda i: (i, 0)),
            pl.BlockSpec((ROW_BLK, MEM_DIM), lambda i: (i, 0)),
            pl.BlockSpec((3 * MEM_DIM, MSG_DIM), lambda i: (0, 0)),
            pl.BlockSpec((3 * MEM_DIM, MEM_DIM), lambda i: (0, 0)),
            pl.BlockSpec((3 * MEM_DIM,), lambda i: (0,)),
            pl.BlockSpec((3 * MEM_DIM,), lambda i: (0,)),
        ],
        out_specs=pl.BlockSpec((ROW_BLK, MEM_DIM), lambda i: (i, 0)),
        out_shape=jax.ShapeDtypeStruct((B_TOTAL, MEM_DIM), jnp.float32),
    )(msgs, h, W_ih, W_hh, b_ih, b_hh)


def kernel(memory, last_update, unique_node_ids, unique_messages, timestamps,
           W_ih, W_hh, b_ih, b_hh):
    ids = unique_node_ids
    h = _sc_gather(memory, ids)
    win = _sc_win(ids)
    upd = _gru(unique_messages, h, W_ih, W_hh, b_ih, b_hh)
    mem_ref = jax.new_ref(_tc_copy(memory, h))
    lu_ref = jax.new_ref(last_update)
    _sc_scatter(mem_ref, lu_ref, ids, win, upd, timestamps)
    return mem_ref[...], lu_ref[...]


# speedup vs baseline: 2.2153x; 1.0102x over previous
import functools

import jax
import jax.numpy as jnp
from jax import lax
from jax.experimental import pallas as pl
from jax.experimental.pallas import tpu as pltpu
from jax.experimental.pallas import tpu_sc as plsc

N_NODES = 100000
MEM_DIM = 128
MSG_DIM = 256
B_TOTAL = 4096
ROW_BLK = 2048
LANES = 16
NC, NS = 2, 16
NW = NC * NS
B_PER_W = B_TOTAL // NW
POS_BITS = 12
SENTINEL = 0x7FFFFFFF
UNROLL = 4

_mesh = plsc.VectorSubcoreMesh(core_axis_name="c", subcore_axis_name="s")


@functools.partial(
    pl.kernel,
    mesh=_mesh,
    out_type=jax.ShapeDtypeStruct((B_TOTAL, MEM_DIM), jnp.float32),
    scratch_types=[
        pltpu.VMEM((B_PER_W,), jnp.int32),
        pltpu.VMEM((B_PER_W, MEM_DIM), jnp.float32),
        pltpu.SemaphoreType.DMA,
    ],
)
def _sc_gather(mem_hbm, idx_hbm, out_hbm, idx_v, rows_v, sem):
    wid = lax.axis_index("s") * NC + lax.axis_index("c")
    base = wid * B_PER_W
    pltpu.sync_copy(idx_hbm.at[pl.ds(base, B_PER_W)], idx_v)
    pltpu.async_copy(mem_hbm.at[idx_v], rows_v, sem).wait()
    pltpu.sync_copy(rows_v, out_hbm.at[pl.ds(base, B_PER_W)])


@functools.partial(
    pl.kernel,
    mesh=_mesh,
    out_type=jax.ShapeDtypeStruct((B_TOTAL,), jnp.int32),
    scratch_types=[
        pltpu.VMEM((B_TOTAL,), jnp.int32),
        pltpu.VMEM((B_TOTAL,), jnp.int32),
        pltpu.VMEM((N_NODES,), jnp.int32),
        pltpu.VMEM((2 * LANES,), jnp.int32),
    ],
    compiler_params=pltpu.CompilerParams(needs_layout_passes=False),
)
def _sc_win(idx_hbm, win_hbm, ids_v, win_v, table_v, shift_v):
    wid = lax.axis_index("s") * NC + lax.axis_index("c")

    @pl.when(wid == 0)
    def _win():
        pltpu.sync_copy(idx_hbm, ids_v)
        shift_v[pl.ds(LANES, LANES)] = jnp.full((LANES,), SENTINEL,
                                                dtype=jnp.int32)
        liota = lax.iota(jnp.int32, LANES)

        def _init(it, carry):
            for u in range(UNROLL):
                ids16 = ids_v[pl.ds((it * UNROLL + u) * LANES, LANES)]
                plsc.store_scatter(table_v, [ids16],
                                   jnp.zeros((LANES,), jnp.int32))
            return carry

        lax.fori_loop(0, B_TOTAL // (LANES * UNROLL), _init, 0)

        def _scan(it, carry):
            for u in range(UNROLL):
                vi = it * UNROLL + u
                ids16 = ids_v[pl.ds(vi * LANES, LANES)]
                pos = liota + vi * LANES
                k = ids16 * (1 << POS_BITS) + pos
                ks, _ = plsc.sort_key_val(k, k)
                shift_v[pl.ds(0, LANES)] = ks
                nxt = plsc.load_gather(shift_v, [liota + 1])
                sid = lax.shift_right_logical(ks, POS_BITS)
                spos = lax.bitwise_and(ks, (1 << POS_BITS) - 1)
                winner = lax.shift_right_logical(nxt, POS_BITS) != sid
                cur = plsc.load_gather(table_v, [sid])
                plsc.store_scatter(table_v, [sid], jnp.maximum(cur, spos),
                                   mask=winner)
            return carry

        lax.fori_loop(0, B_TOTAL // (LANES * UNROLL), _scan, 0)

        def _readout(it, carry):
            for u in range(UNROLL):
                vi = it * UNROLL + u
                ids16 = ids_v[pl.ds(vi * LANES, LANES)]
                win_v[pl.ds(vi * LANES, LANES)] = plsc.load_gather(
                    table_v, [ids16])
            return carry

        lax.fori_loop(0, B_TOTAL // (LANES * UNROLL), _readout, 0)
        pltpu.sync_copy(win_v, win_hbm)


@functools.partial(
    pl.kernel,
    mesh=_mesh,
    out_type=(),
    scratch_types=[
        pltpu.VMEM((B_PER_W,), jnp.int32),
        pltpu.VMEM((B_PER_W,), jnp.int32),
        pltpu.VMEM((B_PER_W, MEM_DIM), jnp.float32),
        pltpu.VMEM((B_PER_W,), jnp.float32),
        pltpu.SemaphoreType.DMA,
        pltpu.SemaphoreType.DMA,
        pltpu.SemaphoreType.DMA,
        pltpu.SemaphoreType.DMA,
    ],
)
def _sc_scatter(mem_ref, lu_ref, idx_hbm, win_hbm, upd_hbm, ts_hbm,
                idx_v, win_v, rows_v, ts_v, sem_a, sem_b, sem_c, sem_d):
    wid = lax.axis_index("s") * NC + lax.axis_index("c")
    base = wid * B_PER_W
    c_idx = pltpu.async_copy(idx_hbm.at[pl.ds(base, B_PER_W)], idx_v, sem_a)
    c_win = pltpu.async_copy(win_hbm.at[pl.ds(base, B_PER_W)], win_v, sem_b)
    c_idx.wait()
    c_win.wait()
    g_rows = pltpu.async_copy(upd_hbm.at[win_v], rows_v, sem_a)
    g_ts = pltpu.async_copy(ts_hbm.at[win_v], ts_v, sem_b)
    g_rows.wait()
    s_rows = pltpu.async_copy(rows_v, mem_ref.at[idx_v], sem_c)
    g_ts.wait()
    s_ts = pltpu.async_copy(ts_v, lu_ref.at[idx_v], sem_d)
    s_rows.wait()
    s_ts.wait()


COPY_BLK = 25000


def _copy_body(dep_ref, src_ref, out_ref):
    out_ref[...] = src_ref[...]


def _tc_copy(memory, h):
    grid = (N_NODES // COPY_BLK,)
    return pl.pallas_call(
        _copy_body,
        grid=grid,
        in_specs=[
            pl.BlockSpec((8, MEM_DIM), lambda i: (0, 0)),
            pl.BlockSpec((COPY_BLK, MEM_DIM), lambda i: (i, 0)),
        ],
        out_specs=pl.BlockSpec((COPY_BLK, MEM_DIM), lambda i: (i, 0)),
        out_shape=jax.ShapeDtypeStruct((N_NODES, MEM_DIM), jnp.float32),
    )(h, memory)


def _gru_body(msg_ref, h_ref, wih_ref, whh_ref, bih_ref, bhh_ref, upd_ref):
    x = msg_ref[...]
    h = h_ref[...]
    gi = lax.dot_general(x, wih_ref[...], (((1,), (1,)), ((), ())),
                         preferred_element_type=jnp.float32) + bih_ref[...]
    gh = lax.dot_general(h, whh_ref[...], (((1,), (1,)), ((), ())),
                         preferred_element_type=jnp.float32) + bhh_ref[...]
    i_r = gi[:, 0 * MEM_DIM:1 * MEM_DIM]
    i_z = gi[:, 1 * MEM_DIM:2 * MEM_DIM]
    i_n = gi[:, 2 * MEM_DIM:3 * MEM_DIM]
    h_r = gh[:, 0 * MEM_DIM:1 * MEM_DIM]
    h_z = gh[:, 1 * MEM_DIM:2 * MEM_DIM]
    h_n = gh[:, 2 * MEM_DIM:3 * MEM_DIM]
    r = jax.nn.sigmoid(i_r + h_r)
    z = jax.nn.sigmoid(i_z + h_z)
    n = jnp.tanh(i_n + r * h_n)
    upd_ref[...] = (1.0 - z) * n + z * h


def _gru(msgs, h, W_ih, W_hh, b_ih, b_hh):
    grid = (B_TOTAL // ROW_BLK,)
    return pl.pallas_call(
        _gru_body,
        grid=grid,
        in_specs=[
            pl.BlockSpec((ROW_BLK, MSG_DIM), lambda i: (i, 0)),
            pl.BlockSpec((ROW_BLK, MEM_DIM), lambda i: (i, 0)),
            pl.BlockSpec((3 * MEM_DIM, MSG_DIM), lambda i: (0, 0)),
            pl.BlockSpec((3 * MEM_DIM, MEM_DIM), lambda i: (0, 0)),
            pl.BlockSpec((3 * MEM_DIM,), lambda i: (0,)),
            pl.BlockSpec((3 * MEM_DIM,), lambda i: (0,)),
        ],
        out_specs=pl.BlockSpec((ROW_BLK, MEM_DIM), lambda i: (i, 0)),
        out_shape=jax.ShapeDtypeStruct((B_TOTAL, MEM_DIM), jnp.float32),
    )(msgs, h, W_ih, W_hh, b_ih, b_hh)


def kernel(memory, last_update, unique_node_ids, unique_messages, timestamps,
           W_ih, W_hh, b_ih, b_hh):
    ids = unique_node_ids
    h = _sc_gather(memory, ids)
    win = _sc_win(ids)
    upd = _gru(unique_messages, h, W_ih, W_hh, b_ih, b_hh)
    mem_ref = jax.new_ref(_tc_copy(memory, h))
    lu_ref = jax.new_ref(last_update)
    _sc_scatter(mem_ref, lu_ref, ids, win, upd, timestamps)
    return mem_ref[...], lu_ref[...]
